# Initial kernel scaffold; baseline (speedup 1.0000x reference)
import jax
import jax.numpy as jnp
from jax.experimental import pallas as pl


def kernel(x, edge_index, W1, a_src1, a_dst1, b1, W2, a_src2, a_dst2, b2, W3, a_src3, a_dst3, b3, W_res, g1, be1, g2, be2):
    raise NotImplementedError("write your pallas kernel here")


# speedup vs baseline: 20.2332x; 20.2332x over previous
import functools

import jax
import jax.numpy as jnp
from jax import lax
from jax.experimental import pallas as pl
from jax.experimental.pallas import tpu as pltpu
from jax.experimental.pallas import tpu_sc as plsc

N = 10000
F_IN = 128
H = 8
C = 16
TH = H * C
OUT = 40
E = 320000
E_TOT = E + N
EPAD = 330752
B = 128
_BN_S = float((1.0 + 1e-5) ** -0.5)


def _tc1_body(x_ref, w1_ref, asf_ref, adf_ref, wres_ref,
              h_ref, als_ref, ald_ref, res_ref):
    x = x_ref[...]
    h = jnp.dot(x, w1_ref[...], preferred_element_type=jnp.float32)
    h_ref[...] = h
    als_ref[...] = jnp.dot(h, asf_ref[...], preferred_element_type=jnp.float32)
    ald_ref[...] = jnp.dot(h, adf_ref[...], preferred_element_type=jnp.float32)
    res_ref[...] = jnp.dot(x, wres_ref[...], preferred_element_type=jnp.float32)


_NB = 1000


def _row(i):
    return (i, 0)


def _rep(i):
    return (0, 0)


def _rep3(i):
    return (0, 0, 0)


def _row3(i):
    return (0, i, 0)


def _tc1(x, W1, Asf, Adf, Wres):
    return pl.pallas_call(
        _tc1_body,
        grid=(N // _NB,),
        in_specs=[
            pl.BlockSpec((_NB, F_IN), _row),
            pl.BlockSpec((F_IN, TH), _rep),
            pl.BlockSpec((F_IN, H), _rep),
            pl.BlockSpec((F_IN, H), _rep),
            pl.BlockSpec((F_IN, TH), _rep),
        ],
        out_specs=[
            pl.BlockSpec((_NB, TH), _row),
            pl.BlockSpec((_NB, H), _row),
            pl.BlockSpec((_NB, H), _row),
            pl.BlockSpec((_NB, TH), _row),
        ],
        out_shape=[
            jax.ShapeDtypeStruct((N, TH), jnp.float32),
            jax.ShapeDtypeStruct((N, H), jnp.float32),
            jax.ShapeDtypeStruct((N, H), jnp.float32),
            jax.ShapeDtypeStruct((N, TH), jnp.float32),
        ],
    )(x, W1, Asf, Adf, Wres)


def _mid_body(acc_ref, den_ref, res_ref, b_ref, g_ref, be_ref,
              w_ref, asf_ref, adf_ref,
              x_ref, h_ref, als_ref, ald_ref):
    acc = acc_ref[0] + acc_ref[1]
    nb = acc.shape[0]
    den = den_ref[0] + den_ref[1]
    denb = jnp.concatenate(
        [jnp.broadcast_to(den[:, h:h + 1], (nb, C)) for h in range(H)], axis=1)
    gat = acc / (denb + 1e-16)
    v = (gat + b_ref[...]) * (g_ref[...] * _BN_S) + be_ref[...] + res_ref[...]
    xn = jnp.where(v > 0, v, jnp.exp(v) - 1.0)
    x_ref[...] = xn
    h2 = jnp.dot(xn, w_ref[...], preferred_element_type=jnp.float32)
    h_ref[...] = h2
    als_ref[...] = jnp.dot(h2, asf_ref[...], preferred_element_type=jnp.float32)
    ald_ref[...] = jnp.dot(h2, adf_ref[...], preferred_element_type=jnp.float32)


def _tc_mid(acc_t, den_t, res, b, g, be, W, Asf, Adf, kout, hout):
    return pl.pallas_call(
        _mid_body,
        grid=(N // _NB,),
        in_specs=[
            pl.BlockSpec((2, _NB, TH), _row3),
            pl.BlockSpec((2, _NB, H), _row3),
            pl.BlockSpec((_NB, TH), _row),
            pl.BlockSpec((1, TH), _rep),
            pl.BlockSpec((1, TH), _rep),
            pl.BlockSpec((1, TH), _rep),
            pl.BlockSpec((TH, kout), _rep),
            pl.BlockSpec((kout, hout), _rep),
            pl.BlockSpec((kout, hout), _rep),
        ],
        out_specs=[
            pl.BlockSpec((_NB, TH), _row),
            pl.BlockSpec((_NB, kout), _row),
            pl.BlockSpec((_NB, hout), _row),
            pl.BlockSpec((_NB, hout), _row),
        ],
        out_shape=[
            jax.ShapeDtypeStruct((N, TH), jnp.float32),
            jax.ShapeDtypeStruct((N, kout), jnp.float32),
            jax.ShapeDtypeStruct((N, hout), jnp.float32),
            jax.ShapeDtypeStruct((N, hout), jnp.float32),
        ],
    )(acc_t, den_t, res, b.reshape(1, TH), g.reshape(1, TH), be.reshape(1, TH),
      W, Asf, Adf)


def _tc4_body(acc_ref, den_ref, b_ref, out_ref):
    a = acc_ref[0] + acc_ref[1]
    den = (den_ref[0, :, 0:1] + den_ref[0, :, 1:2]
           + den_ref[1, :, 0:1] + den_ref[1, :, 1:2])
    blocks = [a[:, 16 * cg:16 * cg + 8] + a[:, 16 * cg + 8:16 * cg + 16]
              for cg in range(5)]
    gat = jnp.concatenate(blocks, axis=1)
    out_ref[...] = gat / (jnp.broadcast_to(den, (a.shape[0], OUT)) + 1e-16) + b_ref[...]


def _tc4(acc_t, den_t, b3):
    return pl.pallas_call(
        _tc4_body,
        grid=(N // _NB,),
        in_specs=[
            pl.BlockSpec((2, _NB, TH), _row3),
            pl.BlockSpec((2, _NB, 2), _row3),
            pl.BlockSpec((1, OUT), _rep),
        ],
        out_specs=pl.BlockSpec((_NB, OUT), _row),
        out_shape=jax.ShapeDtypeStruct((N, OUT), jnp.float32),
    )(acc_t, den_t, b3.reshape(1, OUT))


def _make_sc(ncg):
    nsplit = 32 // ncg
    esz = EPAD // nsplit
    nchunks = esz // B
    mesh = plsc.VectorSubcoreMesh(core_axis_name="c", subcore_axis_name="s")

    @functools.partial(
        pl.kernel,
        out_type=(
            jax.ShapeDtypeStruct((2, 16, N * 8), jnp.float32),
            jax.ShapeDtypeStruct((2, 16, N), jnp.float32),
        ),
        mesh=mesh,
        scratch_types=[
            pltpu.VMEM((N,), jnp.float32),
            pltpu.VMEM((N,), jnp.float32),
            pltpu.VMEM((N * 8,), jnp.float32),
            pltpu.VMEM((N,), jnp.float32),
            pltpu.VMEM((2, 2, B), jnp.int32),
            pltpu.VMEM((2, B), jnp.int32),
            pltpu.VMEM((2, B, 8), jnp.float32),
            pltpu.SemaphoreType.DMA((2,)),
            pltpu.SemaphoreType.DMA((2,)),
            pltpu.SemaphoreType.DMA,
        ],
        compiler_params=pltpu.CompilerParams(use_tc_tiling_on_sc=False,
                                             needs_layout_passes=False),
    )
    def sc_kernel(hfl, alst, aldt, ed, accP, denP,
                  als_v, ald_v, acc_v, den_v, sdbuf, idxbuf, hslbuf,
                  sem_sd, sem_g, sem_m):
        iota16 = lax.broadcasted_iota(jnp.int32, (16,), 0)
        c = lax.axis_index("c")
        s = lax.axis_index("s")
        if ncg == 16:
            cg = s
            head = s // 2
            eoff = c * esz
        else:
            cg = s // 2
            head = 0
            eoff = (c * 2 + s % 2) * esz

        cp1 = pltpu.async_copy(alst.at[head], als_v, sem_m)
        cp2 = pltpu.async_copy(aldt.at[head], ald_v, sem_m)
        cp1.wait()
        cp2.wait()

        zf = jnp.zeros((16,), jnp.float32)

        def zacc(i, _):
            acc_v[pl.ds(i * 16, 16)] = zf
            return _
        lax.fori_loop(0, (N * 8) // 16, zacc, None)

        def zden(i, _):
            den_v[pl.ds(i * 16, 16)] = zf
            return _
        lax.fori_loop(0, N // 16, zden, None)

        def issue_sd(ci, b):
            pltpu.async_copy(ed.at[:, pl.ds(eoff + ci * B, B)],
                             sdbuf.at[b], sem_sd.at[b])

        def wait_sd(ci, b):
            pltpu.make_async_copy(ed.at[:, pl.ds(eoff + ci * B, B)],
                                  sdbuf.at[b], sem_sd.at[b]).wait()

        def prep_gather(b):
            for g in range(B // 16):
                sv = sdbuf[b, 0, pl.ds(g * 16, 16)]
                idxbuf[b, pl.ds(g * 16, 16)] = sv * ncg + cg
            pltpu.async_copy(hfl.at[idxbuf.at[b]], hslbuf.at[b], sem_g.at[b])

        def wait_gather(b):
            pltpu.make_async_copy(hfl.at[idxbuf.at[b]], hslbuf.at[b],
                                  sem_g.at[b]).wait()

        issue_sd(0, 0)
        issue_sd(1, 1)
        wait_sd(0, 0)
        prep_gather(0)

        def step(i, p, q):
            base = eoff + i * B
            wait_gather(p)

            @pl.when(i + 1 < nchunks)
            def _():
                wait_sd(i + 1, q)
                prep_gather(q)

            for g in range(B // 16):
                sv = sdbuf[p, 0, pl.ds(g * 16, 16)]
                dv = sdbuf[p, 1, pl.ds(g * 16, 16)]
                av = plsc.load_gather(als_v, [sv])
                bv = plsc.load_gather(ald_v, [dv])
                e = av + bv
                e = jnp.maximum(e, e * 0.2)
                ex = jnp.exp(e)
                gid = iota16 + (base + g * 16)
                ex = jnp.where(gid < E_TOT, ex, 0.0)
                plsc.addupdate_scatter(den_v, [dv], ex)
                dv8 = dv * 8
                ridx = iota16 + (g * 16)
                for ch in range(8):
                    hv = plsc.load_gather(
                        hslbuf.at[p], [ridx, jnp.full((16,), ch, jnp.int32)])
                    plsc.addupdate_scatter(acc_v, [dv8 + ch], hv * ex)

            @pl.when(i + 2 < nchunks)
            def _():
                issue_sd(i + 2, p)

        def body2(j, _):
            step(2 * j, 0, 1)
            step(2 * j + 1, 1, 0)
            return _
        lax.fori_loop(0, nchunks // 2, body2, None)

        pltpu.sync_copy(acc_v, accP.at[c, s])
        pltpu.sync_copy(den_v, denP.at[c, s])

    return sc_kernel


_sc16 = _make_sc(16)
_sc8 = _make_sc(8)


def _headmat(a):
    hh, cc = a.shape
    eye = jnp.eye(hh, dtype=a.dtype)
    return (a[:, :, None] * eye[:, None, :]).reshape(hh * cc, hh)


def kernel(x, edge_index, W1, a_src1, a_dst1, b1, W2, a_src2, a_dst2, b2,
           W3, a_src3, a_dst3, b3, W_res, g1, be1, g2, be2):
    loops = jnp.arange(N, dtype=edge_index.dtype)
    src = jnp.concatenate([edge_index[0], loops])
    dst = jnp.concatenate([edge_index[1], loops])
    pad = EPAD - E_TOT
    ed = jnp.stack([jnp.pad(src, (0, pad)), jnp.pad(dst, (0, pad))])

    As1f = _headmat(a_src1)
    Ad1f = _headmat(a_dst1)
    As2f = _headmat(a_src2)
    Ad2f = _headmat(a_dst2)
    As3f = jnp.pad(_headmat(a_src3), ((0, 24), (0, 0)))
    Ad3f = jnp.pad(_headmat(a_dst3), ((0, 24), (0, 0)))
    W3p = jnp.pad(W3, ((0, 0), (0, 24)))

    def acc_t(accP):
        return accP.reshape(2, 16, N, 8).transpose(0, 2, 1, 3).reshape(2, N, TH)

    h1, als1, ald1, res1 = _tc1(x, W1, As1f, Ad1f, W_res)
    accP1, denP1 = _sc16(h1.reshape(N * C, 8),
                         als1.T.reshape(H, N), ald1.T.reshape(H, N), ed)
    den1t = denP1[:, 0::2].transpose(0, 2, 1)
    x1, h2, als2, ald2 = _tc_mid(acc_t(accP1), den1t,
                                 res1, b1, g1, be1, W2, As2f, Ad2f, TH, H)
    accP2, denP2 = _sc16(h2.reshape(N * C, 8),
                         als2.T.reshape(H, N), ald2.T.reshape(H, N), ed)
    den2t = denP2[:, 0::2].transpose(0, 2, 1)
    _, h3, als3, ald3 = _tc_mid(acc_t(accP2), den2t,
                                x1, b2, g2, be2, W3p, As3f, Ad3f, 64, 1)
    accP3, denP3 = _sc8(h3.reshape(N * 8, 8),
                        als3.T.reshape(1, N), ald3.T.reshape(1, N), ed)
    den3t = denP3[:, 0:2].transpose(0, 2, 1)
    out = _tc4(acc_t(accP3), den3t, b3)
    return out


# speedup vs baseline: 24.3525x; 1.2036x over previous
import functools

import jax
import jax.numpy as jnp
from jax import lax
from jax.experimental import pallas as pl
from jax.experimental.pallas import tpu as pltpu
from jax.experimental.pallas import tpu_sc as plsc

N = 10000
F_IN = 128
H = 8
C = 16
TH = H * C
OUT = 40
E = 320000
E_TOT = E + N
EPAD = 330752
B = 128
_BN_S = float((1.0 + 1e-5) ** -0.5)


def _tc1_body(x_ref, w1_ref, asf_ref, adf_ref, wres_ref,
              h_ref, als_ref, ald_ref, res_ref):
    x = x_ref[...]
    h = jnp.dot(x, w1_ref[...], preferred_element_type=jnp.float32)
    h_ref[...] = h
    als_ref[...] = jnp.dot(h, asf_ref[...], preferred_element_type=jnp.float32)
    ald_ref[...] = jnp.dot(h, adf_ref[...], preferred_element_type=jnp.float32)
    res_ref[...] = jnp.dot(x, wres_ref[...], preferred_element_type=jnp.float32)


_NB = 1000


def _row(i):
    return (i, 0)


def _rep(i):
    return (0, 0)


def _rep3(i):
    return (0, 0, 0)


def _row3(i):
    return (0, i, 0)


def _tc1(x, W1, Asf, Adf, Wres):
    return pl.pallas_call(
        _tc1_body,
        grid=(N // _NB,),
        in_specs=[
            pl.BlockSpec((_NB, F_IN), _row),
            pl.BlockSpec((F_IN, TH), _rep),
            pl.BlockSpec((F_IN, H), _rep),
            pl.BlockSpec((F_IN, H), _rep),
            pl.BlockSpec((F_IN, TH), _rep),
        ],
        out_specs=[
            pl.BlockSpec((_NB, TH), _row),
            pl.BlockSpec((_NB, H), _row),
            pl.BlockSpec((_NB, H), _row),
            pl.BlockSpec((_NB, TH), _row),
        ],
        out_shape=[
            jax.ShapeDtypeStruct((N, TH), jnp.float32),
            jax.ShapeDtypeStruct((N, H), jnp.float32),
            jax.ShapeDtypeStruct((N, H), jnp.float32),
            jax.ShapeDtypeStruct((N, TH), jnp.float32),
        ],
    )(x, W1, Asf, Adf, Wres)


def _mid_body(acc_ref, den_ref, res_ref, b_ref, g_ref, be_ref,
              w_ref, asf_ref, adf_ref,
              x_ref, h_ref, als_ref, ald_ref):
    acc = acc_ref[0] + acc_ref[1]
    nb = acc.shape[0]
    den = den_ref[0] + den_ref[1]
    denb = jnp.concatenate(
        [jnp.broadcast_to(den[:, h:h + 1], (nb, C)) for h in range(H)], axis=1)
    gat = acc / (denb + 1e-16)
    v = (gat + b_ref[...]) * (g_ref[...] * _BN_S) + be_ref[...] + res_ref[...]
    xn = jnp.where(v > 0, v, jnp.exp(v) - 1.0)
    x_ref[...] = xn
    h2 = jnp.dot(xn, w_ref[...], preferred_element_type=jnp.float32)
    h_ref[...] = h2
    als_ref[...] = jnp.dot(h2, asf_ref[...], preferred_element_type=jnp.float32)
    ald_ref[...] = jnp.dot(h2, adf_ref[...], preferred_element_type=jnp.float32)


def _tc_mid(acc_t, den_t, res, b, g, be, W, Asf, Adf, kout, hout):
    return pl.pallas_call(
        _mid_body,
        grid=(N // _NB,),
        in_specs=[
            pl.BlockSpec((2, _NB, TH), _row3),
            pl.BlockSpec((2, _NB, H), _row3),
            pl.BlockSpec((_NB, TH), _row),
            pl.BlockSpec((1, TH), _rep),
            pl.BlockSpec((1, TH), _rep),
            pl.BlockSpec((1, TH), _rep),
            pl.BlockSpec((TH, kout), _rep),
            pl.BlockSpec((kout, hout), _rep),
            pl.BlockSpec((kout, hout), _rep),
        ],
        out_specs=[
            pl.BlockSpec((_NB, TH), _row),
            pl.BlockSpec((_NB, kout), _row),
            pl.BlockSpec((_NB, hout), _row),
            pl.BlockSpec((_NB, hout), _row),
        ],
        out_shape=[
            jax.ShapeDtypeStruct((N, TH), jnp.float32),
            jax.ShapeDtypeStruct((N, kout), jnp.float32),
            jax.ShapeDtypeStruct((N, hout), jnp.float32),
            jax.ShapeDtypeStruct((N, hout), jnp.float32),
        ],
    )(acc_t, den_t, res, b.reshape(1, TH), g.reshape(1, TH), be.reshape(1, TH),
      W, Asf, Adf)


def _tc4_body(acc_ref, den_ref, b_ref, out_ref):
    a = acc_ref[0] + acc_ref[1]
    den = (den_ref[0, :, 0:1] + den_ref[0, :, 1:2]
           + den_ref[1, :, 0:1] + den_ref[1, :, 1:2])
    blocks = [a[:, 16 * cg:16 * cg + 8] + a[:, 16 * cg + 8:16 * cg + 16]
              for cg in range(5)]
    gat = jnp.concatenate(blocks, axis=1)
    out_ref[...] = gat / (jnp.broadcast_to(den, (a.shape[0], OUT)) + 1e-16) + b_ref[...]


def _tc4(acc_t, den_t, b3):
    return pl.pallas_call(
        _tc4_body,
        grid=(N // _NB,),
        in_specs=[
            pl.BlockSpec((2, _NB, TH), _row3),
            pl.BlockSpec((2, _NB, 2), _row3),
            pl.BlockSpec((1, OUT), _rep),
        ],
        out_specs=pl.BlockSpec((_NB, OUT), _row),
        out_shape=jax.ShapeDtypeStruct((N, OUT), jnp.float32),
    )(acc_t, den_t, b3.reshape(1, OUT))


def _make_sc(ncg):
    nsplit = 32 // ncg
    esz = EPAD // nsplit
    nchunks = esz // B
    mesh = plsc.VectorSubcoreMesh(core_axis_name="c", subcore_axis_name="s")

    @functools.partial(
        pl.kernel,
        out_type=(
            jax.ShapeDtypeStruct((2, 16, N * 8), jnp.float32),
            jax.ShapeDtypeStruct((2, 16, N), jnp.float32),
        ),
        mesh=mesh,
        scratch_types=[
            pltpu.VMEM((N,), jnp.float32),
            pltpu.VMEM((N,), jnp.float32),
            pltpu.VMEM((N * 8,), jnp.float32),
            pltpu.VMEM((N,), jnp.float32),
            pltpu.VMEM((2, 2, B), jnp.int32),
            pltpu.VMEM((2, B), jnp.int32),
            pltpu.VMEM((2, B, 8), jnp.float32),
            pltpu.SemaphoreType.DMA((2,)),
            pltpu.SemaphoreType.DMA((2,)),
            pltpu.SemaphoreType.DMA,
        ],
        compiler_params=pltpu.CompilerParams(use_tc_tiling_on_sc=False,
                                             needs_layout_passes=False),
    )
    def sc_kernel(hfl, alst, aldt, ed, accP, denP,
                  als_v, ald_v, acc_v, den_v, sdbuf, idxbuf, hslbuf,
                  sem_sd, sem_g, sem_m):
        iota16 = lax.broadcasted_iota(jnp.int32, (16,), 0)
        c = lax.axis_index("c")
        s = lax.axis_index("s")
        if ncg == 16:
            cg = s
            head = s // 2
            eoff = c * esz
        else:
            cg = s // 2
            head = 0
            eoff = (c * 2 + s % 2) * esz

        cp1 = pltpu.async_copy(alst.at[head], als_v, sem_m)
        cp2 = pltpu.async_copy(aldt.at[head], ald_v, sem_m)
        cp1.wait()
        cp2.wait()

        zf = jnp.zeros((16,), jnp.float32)

        def zacc(i, _):
            acc_v[pl.ds(i * 16, 16)] = zf
            return _
        lax.fori_loop(0, (N * 8) // 16, zacc, None)

        def zden(i, _):
            den_v[pl.ds(i * 16, 16)] = zf
            return _
        lax.fori_loop(0, N // 16, zden, None)

        def issue_sd(ci, b):
            pltpu.async_copy(ed.at[:, pl.ds(eoff + ci * B, B)],
                             sdbuf.at[b], sem_sd.at[b])

        def wait_sd(ci, b):
            pltpu.make_async_copy(ed.at[:, pl.ds(eoff + ci * B, B)],
                                  sdbuf.at[b], sem_sd.at[b]).wait()

        def prep_gather(b):
            for g in range(B // 16):
                sv = sdbuf[b, 0, pl.ds(g * 16, 16)]
                idxbuf[b, pl.ds(g * 16, 16)] = sv * ncg + cg
            pltpu.async_copy(hfl.at[idxbuf.at[b]], hslbuf.at[b], sem_g.at[b])

        def wait_gather(b):
            pltpu.make_async_copy(hfl.at[idxbuf.at[b]], hslbuf.at[b],
                                  sem_g.at[b]).wait()

        issue_sd(0, 0)
        issue_sd(1, 1)
        wait_sd(0, 0)
        prep_gather(0)

        def step(i, p, q):
            base = eoff + i * B
            wait_gather(p)

            @pl.when(i + 1 < nchunks)
            def _():
                wait_sd(i + 1, q)
                prep_gather(q)

            for g in range(B // 16):
                sv = sdbuf[p, 0, pl.ds(g * 16, 16)]
                dv = sdbuf[p, 1, pl.ds(g * 16, 16)]
                av = plsc.load_gather(als_v, [sv])
                bv = plsc.load_gather(ald_v, [dv])
                e = av + bv
                e = jnp.maximum(e, e * 0.2)
                ex = jnp.exp(e)
                gid = iota16 + (base + g * 16)
                ex = jnp.where(gid < E_TOT, ex, 0.0)
                plsc.addupdate_scatter(den_v, [dv], ex)
                ridx = iota16 + (g * 16)
                for ch in range(8):
                    hv = plsc.load_gather(
                        hslbuf.at[p], [ridx, jnp.full((16,), ch, jnp.int32)])
                    plsc.addupdate_scatter(acc_v, [dv + ch * N], hv * ex)

            @pl.when(i + 2 < nchunks)
            def _():
                issue_sd(i + 2, p)

        def body2(j, _):
            step(2 * j, 0, 1)
            step(2 * j + 1, 1, 0)
            return _
        lax.fori_loop(0, nchunks // 2, body2, None)

        pltpu.sync_copy(acc_v, accP.at[c, s])
        pltpu.sync_copy(den_v, denP.at[c, s])

    return sc_kernel


_sc16 = _make_sc(16)
_sc8 = _make_sc(8)


def _headmat(a):
    hh, cc = a.shape
    eye = jnp.eye(hh, dtype=a.dtype)
    return (a[:, :, None] * eye[:, None, :]).reshape(hh * cc, hh)


def kernel(x, edge_index, W1, a_src1, a_dst1, b1, W2, a_src2, a_dst2, b2,
           W3, a_src3, a_dst3, b3, W_res, g1, be1, g2, be2):
    loops = jnp.arange(N, dtype=edge_index.dtype)
    src = jnp.concatenate([edge_index[0], loops])
    dst = jnp.concatenate([edge_index[1], loops])
    pad = EPAD - E_TOT
    ed = jnp.stack([jnp.pad(src, (0, pad)), jnp.pad(dst, (0, pad))])

    As1f = _headmat(a_src1)
    Ad1f = _headmat(a_dst1)
    As2f = _headmat(a_src2)
    Ad2f = _headmat(a_dst2)
    As3f = jnp.pad(_headmat(a_src3), ((0, 24), (0, 0)))
    Ad3f = jnp.pad(_headmat(a_dst3), ((0, 24), (0, 0)))
    W3p = jnp.pad(W3, ((0, 0), (0, 24)))

    def acc_t(accP):
        return accP.reshape(2, 16, 8, N).transpose(0, 3, 1, 2).reshape(2, N, TH)

    def pad9(h, ng):
        return h.reshape(N * ng, 8)

    h1, als1, ald1, res1 = _tc1(x, W1, As1f, Ad1f, W_res)
    accP1, denP1 = _sc16(pad9(h1, 16),
                         als1.T.reshape(H, N), ald1.T.reshape(H, N), ed)
    den1t = denP1[:, 0::2].transpose(0, 2, 1)
    x1, h2, als2, ald2 = _tc_mid(acc_t(accP1), den1t,
                                 res1, b1, g1, be1, W2, As2f, Ad2f, TH, H)
    accP2, denP2 = _sc16(pad9(h2, 16),
                         als2.T.reshape(H, N), ald2.T.reshape(H, N), ed)
    den2t = denP2[:, 0::2].transpose(0, 2, 1)
    _, h3, als3, ald3 = _tc_mid(acc_t(accP2), den2t,
                                x1, b2, g2, be2, W3p, As3f, Ad3f, 64, 1)
    accP3, denP3 = _sc8(pad9(h3, 8),
                        als3.T.reshape(1, N), ald3.T.reshape(1, N), ed)
    den3t = denP3[:, 0:2].transpose(0, 2, 1)
    out = _tc4(acc_t(accP3), den3t, b3)
    return out


# speedup vs baseline: 25.9719x; 1.0665x over previous
import functools

import jax
import jax.numpy as jnp
from jax import lax
from jax.experimental import pallas as pl
from jax.experimental.pallas import tpu as pltpu
from jax.experimental.pallas import tpu_sc as plsc

N = 10000
F_IN = 128
H = 8
C = 16
TH = H * C
OUT = 40
E = 320000
E_TOT = E + N
EPAD = 331776
B = 512
SUB = 4
NP = 10001
_BN_S = float((1.0 + 1e-5) ** -0.5)


def _tc1_body(x_ref, w1_ref, asf_ref, adf_ref, wres_ref,
              h_ref, als_ref, ald_ref, res_ref):
    x = x_ref[...]
    h = jnp.dot(x, w1_ref[...], preferred_element_type=jnp.float32)
    h_ref[...] = h
    als_ref[...] = jnp.dot(h, asf_ref[...], preferred_element_type=jnp.float32)
    ald_ref[...] = jnp.dot(h, adf_ref[...], preferred_element_type=jnp.float32)
    res_ref[...] = jnp.dot(x, wres_ref[...], preferred_element_type=jnp.float32)


_NB = 1000


def _row(i):
    return (i, 0)


def _rep(i):
    return (0, 0)


def _rep3(i):
    return (0, 0, 0)


def _row3(i):
    return (0, i, 0)


def _tc1(x, W1, Asf, Adf, Wres):
    return pl.pallas_call(
        _tc1_body,
        grid=(N // _NB,),
        in_specs=[
            pl.BlockSpec((_NB, F_IN), _row),
            pl.BlockSpec((F_IN, TH), _rep),
            pl.BlockSpec((F_IN, H), _rep),
            pl.BlockSpec((F_IN, H), _rep),
            pl.BlockSpec((F_IN, TH), _rep),
        ],
        out_specs=[
            pl.BlockSpec((_NB, TH), _row),
            pl.BlockSpec((_NB, H), _row),
            pl.BlockSpec((_NB, H), _row),
            pl.BlockSpec((_NB, TH), _row),
        ],
        out_shape=[
            jax.ShapeDtypeStruct((N, TH), jnp.float32),
            jax.ShapeDtypeStruct((N, H), jnp.float32),
            jax.ShapeDtypeStruct((N, H), jnp.float32),
            jax.ShapeDtypeStruct((N, TH), jnp.float32),
        ],
    )(x, W1, Asf, Adf, Wres)


def _mid_body(acc_ref, den_ref, res_ref, b_ref, g_ref, be_ref,
              w_ref, asf_ref, adf_ref,
              x_ref, h_ref, als_ref, ald_ref):
    acc = acc_ref[0] + acc_ref[1]
    nb = acc.shape[0]
    den = den_ref[0] + den_ref[1]
    denb = jnp.concatenate(
        [jnp.broadcast_to(den[:, h:h + 1], (nb, C)) for h in range(H)], axis=1)
    gat = acc / (denb + 1e-16)
    v = (gat + b_ref[...]) * (g_ref[...] * _BN_S) + be_ref[...] + res_ref[...]
    xn = jnp.where(v > 0, v, jnp.exp(v) - 1.0)
    x_ref[...] = xn
    h2 = jnp.dot(xn, w_ref[...], preferred_element_type=jnp.float32)
    h_ref[...] = h2
    als_ref[...] = jnp.dot(h2, asf_ref[...], preferred_element_type=jnp.float32)
    ald_ref[...] = jnp.dot(h2, adf_ref[...], preferred_element_type=jnp.float32)


def _tc_mid(acc_t, den_t, res, b, g, be, W, Asf, Adf, kout, hout):
    return pl.pallas_call(
        _mid_body,
        grid=(N // _NB,),
        in_specs=[
            pl.BlockSpec((2, _NB, TH), _row3),
            pl.BlockSpec((2, _NB, H), _row3),
            pl.BlockSpec((_NB, TH), _row),
            pl.BlockSpec((1, TH), _rep),
            pl.BlockSpec((1, TH), _rep),
            pl.BlockSpec((1, TH), _rep),
            pl.BlockSpec((TH, kout), _rep),
            pl.BlockSpec((kout, hout), _rep),
            pl.BlockSpec((kout, hout), _rep),
        ],
        out_specs=[
            pl.BlockSpec((_NB, TH), _row),
            pl.BlockSpec((_NB, kout), _row),
            pl.BlockSpec((_NB, hout), _row),
            pl.BlockSpec((_NB, hout), _row),
        ],
        out_shape=[
            jax.ShapeDtypeStruct((N, TH), jnp.float32),
            jax.ShapeDtypeStruct((N, kout), jnp.float32),
            jax.ShapeDtypeStruct((N, hout), jnp.float32),
            jax.ShapeDtypeStruct((N, hout), jnp.float32),
        ],
    )(acc_t, den_t, res, b.reshape(1, TH), g.reshape(1, TH), be.reshape(1, TH),
      W, Asf, Adf)


def _tc4_body(acc_ref, den_ref, b_ref, out_ref):
    a = acc_ref[0] + acc_ref[1]
    den = (den_ref[0, :, 0:1] + den_ref[0, :, 1:2]
           + den_ref[1, :, 0:1] + den_ref[1, :, 1:2])
    blocks = [a[:, 16 * cg:16 * cg + 8] + a[:, 16 * cg + 8:16 * cg + 16]
              for cg in range(5)]
    gat = jnp.concatenate(blocks, axis=1)
    out_ref[...] = gat / (jnp.broadcast_to(den, (a.shape[0], OUT)) + 1e-16) + b_ref[...]


def _tc4(acc_t, den_t, b3):
    return pl.pallas_call(
        _tc4_body,
        grid=(N // _NB,),
        in_specs=[
            pl.BlockSpec((2, _NB, TH), _row3),
            pl.BlockSpec((2, _NB, 2), _row3),
            pl.BlockSpec((1, OUT), _rep),
        ],
        out_specs=pl.BlockSpec((_NB, OUT), _row),
        out_shape=jax.ShapeDtypeStruct((N, OUT), jnp.float32),
    )(acc_t, den_t, b3.reshape(1, OUT))


def _make_sc(ncg):
    nsplit = 32 // ncg
    esz = EPAD // nsplit
    nchunks = esz // B
    mesh = plsc.VectorSubcoreMesh(core_axis_name="c", subcore_axis_name="s")

    @functools.partial(
        pl.kernel,
        out_type=(
            jax.ShapeDtypeStruct((2, 16, 8 * NP), jnp.float32),
            jax.ShapeDtypeStruct((2, 16, N), jnp.float32),
        ),
        mesh=mesh,
        scratch_types=[
            pltpu.VMEM((N,), jnp.float32),
            pltpu.VMEM((N,), jnp.float32),
            pltpu.VMEM((8 * NP,), jnp.float32),
            pltpu.VMEM((N,), jnp.float32),
            pltpu.VMEM((2, 2, B), jnp.int32),
            pltpu.VMEM((2, SUB, 128), jnp.int32),
            pltpu.VMEM((2, SUB, 128, 8), jnp.float32),
            pltpu.SemaphoreType.DMA((2,)),
            pltpu.SemaphoreType.DMA((2,)),
            pltpu.SemaphoreType.DMA,
        ],
        compiler_params=pltpu.CompilerParams(use_tc_tiling_on_sc=False,
                                             needs_layout_passes=False),
    )
    def sc_kernel(hfl, alst, aldt, ed, accP, denP,
                  als_v, ald_v, acc_v, den_v, sdbuf, idxbuf, hslbuf,
                  sem_sd, sem_g, sem_m):
        iota16 = lax.broadcasted_iota(jnp.int32, (16,), 0)
        patbase = iota16 // 8
        colpat = iota16 % 8
        chpat = colpat * NP
        c = lax.axis_index("c")
        s = lax.axis_index("s")
        if ncg == 16:
            cg = s
            head = s // 2
            eoff = c * esz
        else:
            cg = s // 2
            head = 0
            eoff = (c * 2 + s % 2) * esz

        cp1 = pltpu.async_copy(alst.at[head], als_v, sem_m)
        cp2 = pltpu.async_copy(aldt.at[head], ald_v, sem_m)
        cp1.wait()
        cp2.wait()

        zf = jnp.zeros((16,), jnp.float32)

        def zacc(i, _):
            acc_v[pl.ds(i * 16, 16)] = zf
            return _
        lax.fori_loop(0, (8 * NP - 8) // 16, zacc, None)
        acc_v[pl.ds(8 * NP - 16, 16)] = zf

        def zden(i, _):
            den_v[pl.ds(i * 16, 16)] = zf
            return _
        lax.fori_loop(0, N // 16, zden, None)

        def issue_sd(ci, b):
            pltpu.async_copy(ed.at[:, pl.ds(eoff + ci * B, B)],
                             sdbuf.at[b], sem_sd.at[b])

        def wait_sd(ci, b):
            pltpu.make_async_copy(ed.at[:, pl.ds(eoff + ci * B, B)],
                                  sdbuf.at[b], sem_sd.at[b]).wait()

        def prep_gather(b):
            for g in range(B // 16):
                sv = sdbuf[b, 0, pl.ds(g * 16, 16)]
                idxbuf[b, g // 8, pl.ds((g % 8) * 16, 16)] = sv * ncg + cg
            for j in range(SUB):
                pltpu.async_copy(hfl.at[idxbuf.at[b, j]], hslbuf.at[b, j],
                                 sem_g.at[b])

        def wait_gather(b):
            for j in range(SUB):
                pltpu.make_async_copy(hfl.at[idxbuf.at[b, j]],
                                      hslbuf.at[b, j], sem_g.at[b]).wait()

        issue_sd(0, 0)
        issue_sd(1, 1)
        wait_sd(0, 0)
        prep_gather(0)

        def step(i, p, q):
            base = eoff + i * B
            wait_gather(p)

            @pl.when(i + 1 < nchunks)
            def _():
                wait_sd(i + 1, q)
                prep_gather(q)

            for j in range(SUB):
                hsl2 = hslbuf.at[p, j]
                for gl in range(8):
                    g = j * 8 + gl
                    sv = sdbuf[p, 0, pl.ds(g * 16, 16)]
                    dv = sdbuf[p, 1, pl.ds(g * 16, 16)]
                    av = plsc.load_gather(als_v, [sv])
                    bv = plsc.load_gather(ald_v, [dv])
                    e = av + bv
                    e = jnp.maximum(e, e * 0.2)
                    ex = jnp.exp(e)
                    gid = iota16 + (base + g * 16)
                    ex = jnp.where(gid < E_TOT, ex, 0.0)
                    plsc.addupdate_scatter(den_v, [dv], ex)
                    for pr in range(8):
                        pat = patbase + 2 * pr
                        hvp = plsc.load_gather(hsl2, [gl * 16 + pat, colpat])
                        exb = ex.at[pat].get(mode="promise_in_bounds")
                        dvb = dv.at[pat].get(mode="promise_in_bounds")
                        plsc.addupdate_scatter(acc_v, [dvb + chpat], hvp * exb)

            @pl.when(i + 2 < nchunks)
            def _():
                issue_sd(i + 2, p)

        def body2(j, _):
            step(2 * j, 0, 1)
            step(2 * j + 1, 1, 0)
            return _
        lax.fori_loop(0, nchunks // 2, body2, None)

        pltpu.sync_copy(acc_v, accP.at[c, s])
        pltpu.sync_copy(den_v, denP.at[c, s])

    return sc_kernel


_sc16 = _make_sc(16)
_sc8 = _make_sc(8)


def _headmat(a):
    hh, cc = a.shape
    eye = jnp.eye(hh, dtype=a.dtype)
    return (a[:, :, None] * eye[:, None, :]).reshape(hh * cc, hh)


def kernel(x, edge_index, W1, a_src1, a_dst1, b1, W2, a_src2, a_dst2, b2,
           W3, a_src3, a_dst3, b3, W_res, g1, be1, g2, be2):
    loops = jnp.arange(N, dtype=edge_index.dtype)
    src = jnp.concatenate([edge_index[0], loops])
    dst = jnp.concatenate([edge_index[1], loops])
    pad = EPAD - E_TOT
    ed = jnp.stack([jnp.pad(src, (0, pad)), jnp.pad(dst, (0, pad))])

    As1f = _headmat(a_src1)
    Ad1f = _headmat(a_dst1)
    As2f = _headmat(a_src2)
    Ad2f = _headmat(a_dst2)
    As3f = jnp.pad(_headmat(a_src3), ((0, 24), (0, 0)))
    Ad3f = jnp.pad(_headmat(a_dst3), ((0, 24), (0, 0)))
    W3p = jnp.pad(W3, ((0, 0), (0, 24)))

    def acc_t(accP):
        return accP.reshape(2, 16, 8, NP)[..., :N].transpose(0, 3, 1, 2
                                                             ).reshape(2, N, TH)

    def pad9(h, ng):
        return h.reshape(N * ng, 8)

    h1, als1, ald1, res1 = _tc1(x, W1, As1f, Ad1f, W_res)
    accP1, denP1 = _sc16(pad9(h1, 16),
                         als1.T.reshape(H, N), ald1.T.reshape(H, N), ed)
    den1t = denP1[:, 0::2].transpose(0, 2, 1)
    x1, h2, als2, ald2 = _tc_mid(acc_t(accP1), den1t,
                                 res1, b1, g1, be1, W2, As2f, Ad2f, TH, H)
    accP2, denP2 = _sc16(pad9(h2, 16),
                         als2.T.reshape(H, N), ald2.T.reshape(H, N), ed)
    den2t = denP2[:, 0::2].transpose(0, 2, 1)
    _, h3, als3, ald3 = _tc_mid(acc_t(accP2), den2t,
                                x1, b2, g2, be2, W3p, As3f, Ad3f, 64, 1)
    accP3, denP3 = _sc8(pad9(h3, 8),
                        als3.T.reshape(1, N), ald3.T.reshape(1, N), ed)
    den3t = denP3[:, 0:2].transpose(0, 2, 1)
    out = _tc4(acc_t(accP3), den3t, b3)
    return out
